# baseline (speedup 1.0000x reference)
import functools

import jax
import jax.numpy as jnp
from jax import lax
from jax.experimental import pallas as pl
from jax.experimental.pallas import tpu as pltpu
from jax.experimental.pallas import tpu_sc as plsc

VEC = 64
B = 4096
CTX = 20
NSAMP = 10
NC, NS = 2, 16
NW = NC * NS
BPW = B // NW
LANES = 16
NV = VEC // LANES


_TR_BLK = 1024


def _tr_body(o_ref, ot_ref):
    ot_ref[...] = o_ref[...].T


def _transpose_outputs(outputs):
    n = outputs.shape[1]
    nblk = pl.cdiv(n, _TR_BLK)
    return pl.pallas_call(
        _tr_body,
        grid=(nblk,),
        in_specs=[pl.BlockSpec((VEC, _TR_BLK), lambda i: (0, i))],
        out_specs=pl.BlockSpec((_TR_BLK, VEC), lambda i: (i, 0)),
        out_shape=jax.ShapeDtypeStruct((nblk * _TR_BLK, VEC), jnp.float32),
    )(outputs)


_DW = 4


def _doc_body(pt_hbm, doc_f, out_hbm, doc_idx, slab0, slab1, docbuf, sem):
    wid = lax.axis_index("s") * NC + lax.axis_index("c")
    pltpu.sync_copy(doc_f.at[pl.ds(wid * BPW, BPW)], doc_idx)

    lane = lax.iota(jnp.int32, LANES)
    nwaves = BPW // _DW
    slabs = [slab0, slab1]
    pend = [None, None]

    def fire(w):
        jv = doc_idx[pl.ds((w * _DW // LANES) * LANES, LANES)]
        buf = slabs[w % 2]
        cps = []
        for t in range(_DW):
            off = pl.multiple_of(
                (jv[(w * _DW + t) % LANES] >> 7) << 7, 128)
            cps.append(pltpu.async_copy(pt_hbm.at[:, pl.ds(off, 128)],
                                        buf.at[:, pl.ds(t * 128, 128)], sem))
        return cps

    def extract(w):
        jv = doc_idx[pl.ds((w * _DW // LANES) * LANES, LANES)]
        buf = slabs[w % 2]
        for t in range(_DW):
            sid = jv[(w * _DW + t) % LANES]
            col = jnp.full((LANES,), 0, jnp.int32) + (t * 128 + (sid & 127))
            dstcol = jnp.full((LANES,), w * _DW + t, jnp.int32)
            for k in range(NV):
                rows = k * LANES + lane
                v = plsc.load_gather(buf, [rows, col])
                plsc.store_scatter(docbuf, [rows, dstcol], v)

    pend[0] = fire(0)
    for w in range(nwaves):
        if w + 1 < nwaves:
            pend[(w + 1) % 2] = fire(w + 1)
        for cp in pend[w % 2]:
            cp.wait()
        extract(w)

    pltpu.sync_copy(docbuf, out_hbm.at[wid])


@functools.partial(
    pl.kernel,
    out_type=jax.ShapeDtypeStruct((NW, VEC, BPW), jnp.float32),
    mesh=plsc.VectorSubcoreMesh(core_axis_name="c", subcore_axis_name="s"),
    scratch_types=[
        pltpu.VMEM((BPW,), jnp.int32),
        pltpu.VMEM((VEC, _DW * 128), jnp.float32),
        pltpu.VMEM((VEC, _DW * 128), jnp.float32),
        pltpu.VMEM((VEC, BPW), jnp.float32),
        pltpu.SemaphoreType.DMA,
    ],
    compiler_params=pltpu.CompilerParams(needs_layout_passes=False,
                                         use_tc_tiling_on_sc=False),
)
def _doc_kernel(pt_hbm, doc_f, out_hbm, *scratch):
    _doc_body(pt_hbm, doc_f, out_hbm, *scratch)


_CTX_CHUNK = 64
_N_CTX_CHUNKS = BPW // _CTX_CHUNK
_CTX_ROWS = _CTX_CHUNK * CTX


def _sc_body(ctx_f, samp_f, docrows_hbm, w_hbm, ot_hbm, out_hbm,
             ctx_idx, samp_idx, big, doc_t, inp, res, sem):
    wid = lax.axis_index("s") * NC + lax.axis_index("c")

    pltpu.sync_copy(ctx_f.at[pl.ds(wid * (CTX * BPW), CTX * BPW)],
                    ctx_idx)
    pltpu.sync_copy(samp_f.at[pl.ds(wid * (NSAMP * BPW), NSAMP * BPW)],
                    samp_idx)

    pltpu.sync_copy(docrows_hbm.at[wid], doc_t)

    for h in range(_N_CTX_CHUNKS):
        cps = [
            pltpu.async_copy(w_hbm.at[ctx_idx.at[pl.ds((h * 10 + j) * 128, 128)]],
                             big.at[pl.ds(j * 128, 128)], sem)
            for j in range(10)
        ]
        for cp in cps:
            cp.wait()

        def chunk_body(i, _, h=h):
            p0 = i * CTX
            acc = [big[p0, pl.ds(k * LANES, LANES)] for k in range(NV)]

            def ctx_body(c, acc):
                return tuple(acc[k] + big[p0 + c, pl.ds(k * LANES, LANES)]
                             for k in range(NV))

            acc = lax.fori_loop(1, CTX, ctx_body, tuple(acc))
            gi = h * _CTX_CHUNK + i
            lane = lax.iota(jnp.int32, LANES)
            icol = jnp.full((LANES,), 0, jnp.int32) + gi
            for k in range(NV):
                docg = plsc.load_gather(doc_t, [k * LANES + lane, icol])
                inp[gi, pl.ds(k * LANES, LANES)] = acc[k] + docg
            return 0

        lax.fori_loop(0, _CTX_CHUNK, chunk_body, 0)

    cps = [
        pltpu.async_copy(ot_hbm.at[samp_idx.at[pl.ds(j * 128, 128)]],
                         big.at[pl.ds(j * 128, 128)], sem)
        for j in range(10)
    ]
    for cp in cps:
        cp.wait()

    def score_body(i, _):
        lane = lax.iota(jnp.int32, LANES)
        smask = lane < NSAMP
        rows = i * NSAMP + lane
        vin = [inp[i, pl.ds(k * LANES, LANES)] for k in range(NV)]
        acc = jnp.zeros((LANES,), jnp.float32)
        for d in range(VEC):
            col = jnp.full((LANES,), d, jnp.int32)
            g = plsc.load_gather(big, [rows, col], mask=smask)
            acc = acc + vin[d // LANES][d % LANES] * g
        plsc.store_compressed(res.at[pl.ds(i * NSAMP, LANES)], acc, mask=smask)
        return 0

    lax.fori_loop(0, BPW, score_body, 0)

    pltpu.sync_copy(res.at[pl.ds(0, BPW * NSAMP)],
                    out_hbm.at[pl.ds(wid * (BPW * NSAMP), BPW * NSAMP)])


@functools.partial(
    pl.kernel,
    out_type=jax.ShapeDtypeStruct((B * NSAMP,), jnp.float32),
    mesh=plsc.VectorSubcoreMesh(core_axis_name="c", subcore_axis_name="s"),
    scratch_types=[
        pltpu.VMEM((CTX * BPW,), jnp.int32),
        pltpu.VMEM((NSAMP * BPW,), jnp.int32),
        pltpu.VMEM((_CTX_ROWS + 8, VEC), jnp.float32),
        pltpu.VMEM((VEC, BPW), jnp.float32),
        pltpu.VMEM((BPW, VEC), jnp.float32),
        pltpu.VMEM((BPW * NSAMP + 8, ), jnp.float32),
        pltpu.SemaphoreType.DMA,
    ],
    compiler_params=pltpu.CompilerParams(needs_layout_passes=False,
                                         use_tc_tiling_on_sc=False),
)
def _sc_kernel(ctx_f, samp_f, docrows_hbm, w_hbm, ot_hbm, out_hbm, *scratch):
    _sc_body(ctx_f, samp_f, docrows_hbm, w_hbm, ot_hbm, out_hbm, *scratch)


def kernel(doc_ids, context_ids, sample_ids, paragraph_matrix, word_matrix,
           outputs):
    ot = _transpose_outputs(outputs)
    doc_f = doc_ids.astype(jnp.int32).reshape(B)
    ctx_f = context_ids.astype(jnp.int32).reshape(B * CTX)
    samp_f = sample_ids.astype(jnp.int32).reshape(B * NSAMP)
    pt = paragraph_matrix.T
    docrows = _doc_kernel(pt, doc_f)
    res = _sc_kernel(ctx_f, samp_f, docrows, word_matrix, ot)
    return res.reshape(B, NSAMP)


# speedup vs baseline: 20.2604x; 20.2604x over previous
import functools

import jax
import jax.numpy as jnp
from jax import lax
from jax.experimental import pallas as pl
from jax.experimental.pallas import tpu as pltpu
from jax.experimental.pallas import tpu_sc as plsc

VEC = 64
B = 4096
CTX = 20
NSAMP = 10
NC, NS = 2, 16
NW = NC * NS
BPW = B // NW
LANES = 16
NV = VEC // LANES


_TR_BLK = 1024


def _tr_body(o_ref, ot_ref):
    ot_ref[...] = o_ref[...].T


def _transpose_outputs(outputs):
    n = outputs.shape[1]
    nblk = pl.cdiv(n, _TR_BLK)
    return pl.pallas_call(
        _tr_body,
        grid=(nblk,),
        in_specs=[pl.BlockSpec((VEC, _TR_BLK), lambda i: (0, i))],
        out_specs=pl.BlockSpec((_TR_BLK, VEC), lambda i: (i, 0)),
        out_shape=jax.ShapeDtypeStruct((nblk * _TR_BLK, VEC), jnp.float32),
    )(outputs)


_DW = 4


def _doc_body(pt_hbm, doc_f, out_hbm, doc_idx, slab0, slab1, docbuf, sem):
    wid = lax.axis_index("s") * NC + lax.axis_index("c")
    pltpu.sync_copy(doc_f.at[pl.ds(wid * BPW, BPW)], doc_idx)

    lane = lax.iota(jnp.int32, LANES)
    nwaves = BPW // _DW
    slabs = [slab0, slab1]
    pend = [None, None]

    def fire(w):
        jv = doc_idx[pl.ds((w * _DW // LANES) * LANES, LANES)]
        buf = slabs[w % 2]
        cps = []
        for t in range(_DW):
            off = pl.multiple_of(
                (jv[(w * _DW + t) % LANES] >> 7) << 7, 128)
            cps.append(pltpu.async_copy(pt_hbm.at[:, pl.ds(off, 128)],
                                        buf.at[:, pl.ds(t * 128, 128)], sem))
        return cps

    def extract(w):
        jv = doc_idx[pl.ds((w * _DW // LANES) * LANES, LANES)]
        buf = slabs[w % 2]
        for t in range(_DW):
            sid = jv[(w * _DW + t) % LANES]
            col = jnp.full((LANES,), 0, jnp.int32) + (t * 128 + (sid & 127))
            dstcol = jnp.full((LANES,), w * _DW + t, jnp.int32)
            for k in range(NV):
                rows = k * LANES + lane
                v = plsc.load_gather(buf, [rows, col])
                plsc.store_scatter(docbuf, [rows, dstcol], v)

    pend[0] = fire(0)
    for w in range(nwaves):
        if w + 1 < nwaves:
            pend[(w + 1) % 2] = fire(w + 1)
        for cp in pend[w % 2]:
            cp.wait()
        extract(w)

    pltpu.sync_copy(docbuf, out_hbm.at[wid])


@functools.partial(
    pl.kernel,
    out_type=jax.ShapeDtypeStruct((NW, VEC, BPW), jnp.float32),
    mesh=plsc.VectorSubcoreMesh(core_axis_name="c", subcore_axis_name="s"),
    scratch_types=[
        pltpu.VMEM((BPW,), jnp.int32),
        pltpu.VMEM((VEC, _DW * 128), jnp.float32),
        pltpu.VMEM((VEC, _DW * 128), jnp.float32),
        pltpu.VMEM((VEC, BPW), jnp.float32),
        pltpu.SemaphoreType.DMA,
    ],
    compiler_params=pltpu.CompilerParams(needs_layout_passes=False,
                                         use_tc_tiling_on_sc=True),
)
def _doc_kernel(pt_hbm, doc_f, out_hbm, *scratch):
    _doc_body(pt_hbm, doc_f, out_hbm, *scratch)


_CTX_CHUNK = 64
_N_CTX_CHUNKS = BPW // _CTX_CHUNK
_CTX_ROWS = _CTX_CHUNK * CTX


def _sc_body(ctx_f, samp_f, docrows_hbm, w_hbm, ot_hbm, out_hbm,
             ctx_idx, samp_idx, big, doc_t, inp, res, sem):
    wid = lax.axis_index("s") * NC + lax.axis_index("c")

    pltpu.sync_copy(ctx_f.at[pl.ds(wid * (CTX * BPW), CTX * BPW)],
                    ctx_idx)
    pltpu.sync_copy(samp_f.at[pl.ds(wid * (NSAMP * BPW), NSAMP * BPW)],
                    samp_idx)

    pltpu.sync_copy(docrows_hbm.at[wid], doc_t)

    for h in range(_N_CTX_CHUNKS):
        cps = [
            pltpu.async_copy(w_hbm.at[ctx_idx.at[pl.ds((h * 10 + j) * 128, 128)]],
                             big.at[pl.ds(j * 128, 128)], sem)
            for j in range(10)
        ]
        for cp in cps:
            cp.wait()

        def chunk_body(i, _, h=h):
            p0 = i * CTX
            acc = [big[p0, pl.ds(k * LANES, LANES)] for k in range(NV)]

            def ctx_body(c, acc):
                return tuple(acc[k] + big[p0 + c, pl.ds(k * LANES, LANES)]
                             for k in range(NV))

            acc = lax.fori_loop(1, CTX, ctx_body, tuple(acc))
            gi = h * _CTX_CHUNK + i
            lane = lax.iota(jnp.int32, LANES)
            icol = jnp.full((LANES,), 0, jnp.int32) + gi
            for k in range(NV):
                docg = plsc.load_gather(doc_t, [k * LANES + lane, icol])
                inp[gi, pl.ds(k * LANES, LANES)] = acc[k] + docg
            return 0

        lax.fori_loop(0, _CTX_CHUNK, chunk_body, 0)

    cps = [
        pltpu.async_copy(ot_hbm.at[samp_idx.at[pl.ds(j * 128, 128)]],
                         big.at[pl.ds(j * 128, 128)], sem)
        for j in range(10)
    ]
    for cp in cps:
        cp.wait()

    def score_body(i, _):
        lane = lax.iota(jnp.int32, LANES)
        smask = lane < NSAMP
        rows = i * NSAMP + lane
        vin = [inp[i, pl.ds(k * LANES, LANES)] for k in range(NV)]
        acc = jnp.zeros((LANES,), jnp.float32)
        for d in range(VEC):
            col = jnp.full((LANES,), d, jnp.int32)
            g = plsc.load_gather(big, [rows, col], mask=smask)
            acc = acc + vin[d // LANES][d % LANES] * g
        plsc.store_compressed(res.at[pl.ds(i * NSAMP, LANES)], acc, mask=smask)
        return 0

    lax.fori_loop(0, BPW, score_body, 0)

    pltpu.sync_copy(res.at[pl.ds(0, BPW * NSAMP)],
                    out_hbm.at[pl.ds(wid * (BPW * NSAMP), BPW * NSAMP)])


@functools.partial(
    pl.kernel,
    out_type=jax.ShapeDtypeStruct((B * NSAMP,), jnp.float32),
    mesh=plsc.VectorSubcoreMesh(core_axis_name="c", subcore_axis_name="s"),
    scratch_types=[
        pltpu.VMEM((CTX * BPW,), jnp.int32),
        pltpu.VMEM((NSAMP * BPW,), jnp.int32),
        pltpu.VMEM((_CTX_ROWS + 8, VEC), jnp.float32),
        pltpu.VMEM((VEC, BPW), jnp.float32),
        pltpu.VMEM((BPW, VEC), jnp.float32),
        pltpu.VMEM((BPW * NSAMP + 8, ), jnp.float32),
        pltpu.SemaphoreType.DMA,
    ],
    compiler_params=pltpu.CompilerParams(needs_layout_passes=False,
                                         use_tc_tiling_on_sc=False),
)
def _sc_kernel(ctx_f, samp_f, docrows_hbm, w_hbm, ot_hbm, out_hbm, *scratch):
    _sc_body(ctx_f, samp_f, docrows_hbm, w_hbm, ot_hbm, out_hbm, *scratch)


def kernel(doc_ids, context_ids, sample_ids, paragraph_matrix, word_matrix,
           outputs):
    ot = _transpose_outputs(outputs)
    doc_f = doc_ids.astype(jnp.int32).reshape(B)
    ctx_f = context_ids.astype(jnp.int32).reshape(B * CTX)
    samp_f = sample_ids.astype(jnp.int32).reshape(B * NSAMP)
    pt = paragraph_matrix.T
    docrows = _doc_kernel(pt, doc_f)
    res = _sc_kernel(ctx_f, samp_f, docrows, word_matrix, ot)
    return res.reshape(B, NSAMP)


# speedup vs baseline: 23.9273x; 1.1810x over previous
import functools

import jax
import jax.numpy as jnp
from jax import lax
from jax.experimental import pallas as pl
from jax.experimental.pallas import tpu as pltpu
from jax.experimental.pallas import tpu_sc as plsc

VEC = 64
B = 4096
CTX = 20
NSAMP = 10
NC, NS = 2, 16
NW = NC * NS
BPW = B // NW
LANES = 16
NV = VEC // LANES


_DW = 4


def _doc_body(pt_hbm, doc_f, out_hbm, doc_idx, slab0, slab1, docbuf, sem):
    wid = lax.axis_index("s") * NC + lax.axis_index("c")
    pltpu.sync_copy(doc_f.at[pl.ds(wid * BPW, BPW)], doc_idx)

    lane = lax.iota(jnp.int32, LANES)
    nwaves = BPW // _DW
    slabs = [slab0, slab1]
    pend = [None, None]

    def fire(w):
        jv = doc_idx[pl.ds((w * _DW // LANES) * LANES, LANES)]
        buf = slabs[w % 2]
        cps = []
        for t in range(_DW):
            off = pl.multiple_of(
                (jv[(w * _DW + t) % LANES] >> 7) << 7, 128)
            cps.append(pltpu.async_copy(pt_hbm.at[:, pl.ds(off, 128)],
                                        buf.at[:, pl.ds(t * 128, 128)], sem))
        return cps

    def extract(w):
        jv = doc_idx[pl.ds((w * _DW // LANES) * LANES, LANES)]
        buf = slabs[w % 2]
        for t in range(_DW):
            sid = jv[(w * _DW + t) % LANES]
            col = jnp.full((LANES,), 0, jnp.int32) + (t * 128 + (sid & 127))
            dstcol = jnp.full((LANES,), w * _DW + t, jnp.int32)
            for k in range(NV):
                rows = k * LANES + lane
                v = plsc.load_gather(buf, [rows, col])
                plsc.store_scatter(docbuf, [rows, dstcol], v)

    pend[0] = fire(0)
    for w in range(nwaves):
        if w + 1 < nwaves:
            pend[(w + 1) % 2] = fire(w + 1)
        for cp in pend[w % 2]:
            cp.wait()
        extract(w)

    pltpu.sync_copy(docbuf, out_hbm.at[wid])


@functools.partial(
    pl.kernel,
    out_type=jax.ShapeDtypeStruct((NW, VEC, BPW), jnp.float32),
    mesh=plsc.VectorSubcoreMesh(core_axis_name="c", subcore_axis_name="s"),
    scratch_types=[
        pltpu.VMEM((BPW,), jnp.int32),
        pltpu.VMEM((VEC, _DW * 128), jnp.float32),
        pltpu.VMEM((VEC, _DW * 128), jnp.float32),
        pltpu.VMEM((VEC, BPW), jnp.float32),
        pltpu.SemaphoreType.DMA,
    ],
    compiler_params=pltpu.CompilerParams(needs_layout_passes=False,
                                         use_tc_tiling_on_sc=True),
)
def _doc_kernel(pt_hbm, doc_f, out_hbm, *scratch):
    _doc_body(pt_hbm, doc_f, out_hbm, *scratch)


_CTX_CHUNK = 64
_N_CTX_CHUNKS = BPW // _CTX_CHUNK
_CTX_ROWS = _CTX_CHUNK * CTX


def _sc_body(ctx_f, samp_f, docrows_hbm, w_hbm, ot_hbm, out_hbm,
             ctx_idx, samp_idx, big, doc_t, inp, res, sem):
    wid = lax.axis_index("s") * NC + lax.axis_index("c")

    pltpu.sync_copy(ctx_f.at[pl.ds(wid * (CTX * BPW), CTX * BPW)],
                    ctx_idx)
    pltpu.sync_copy(samp_f.at[pl.ds(wid * (NSAMP * BPW), NSAMP * BPW)],
                    samp_idx)

    pltpu.sync_copy(docrows_hbm.at[wid], doc_t)

    for h in range(_N_CTX_CHUNKS):
        cps = [
            pltpu.async_copy(w_hbm.at[ctx_idx.at[pl.ds((h * 10 + j) * 128, 128)]],
                             big.at[pl.ds(j * 128, 128)], sem)
            for j in range(10)
        ]
        for cp in cps:
            cp.wait()

        def chunk_body(i, _, h=h):
            p0 = i * CTX
            acc = [big[p0, pl.ds(k * LANES, LANES)] for k in range(NV)]

            def ctx_body(c, acc):
                return tuple(acc[k] + big[p0 + c, pl.ds(k * LANES, LANES)]
                             for k in range(NV))

            acc = lax.fori_loop(1, CTX, ctx_body, tuple(acc))
            gi = h * _CTX_CHUNK + i
            lane = lax.iota(jnp.int32, LANES)
            icol = jnp.full((LANES,), 0, jnp.int32) + gi
            for k in range(NV):
                docg = plsc.load_gather(doc_t, [k * LANES + lane, icol])
                inp[gi, pl.ds(k * LANES, LANES)] = acc[k] + docg
            return 0

        lax.fori_loop(0, _CTX_CHUNK, chunk_body, 0)

    cps = [
        pltpu.async_copy(ot_hbm.at[samp_idx.at[pl.ds(j * 128, 128)]],
                         big.at[pl.ds(j * 128, 128)], sem)
        for j in range(10)
    ]
    for cp in cps:
        cp.wait()

    def score_body(i, _):
        lane = lax.iota(jnp.int32, LANES)
        smask = lane < NSAMP
        rows = i * NSAMP + lane
        vin = [inp[i, pl.ds(k * LANES, LANES)] for k in range(NV)]
        acc = jnp.zeros((LANES,), jnp.float32)
        for d in range(VEC):
            col = jnp.full((LANES,), d, jnp.int32)
            g = plsc.load_gather(big, [rows, col], mask=smask)
            acc = acc + vin[d // LANES][d % LANES] * g
        plsc.store_compressed(res.at[pl.ds(i * NSAMP, LANES)], acc, mask=smask)
        return 0

    lax.fori_loop(0, BPW, score_body, 0)

    pltpu.sync_copy(res.at[pl.ds(0, BPW * NSAMP)],
                    out_hbm.at[pl.ds(wid * (BPW * NSAMP), BPW * NSAMP)])


@functools.partial(
    pl.kernel,
    out_type=jax.ShapeDtypeStruct((B * NSAMP,), jnp.float32),
    mesh=plsc.VectorSubcoreMesh(core_axis_name="c", subcore_axis_name="s"),
    scratch_types=[
        pltpu.VMEM((CTX * BPW,), jnp.int32),
        pltpu.VMEM((NSAMP * BPW,), jnp.int32),
        pltpu.VMEM((_CTX_ROWS + 8, VEC), jnp.float32),
        pltpu.VMEM((VEC, BPW), jnp.float32),
        pltpu.VMEM((BPW, VEC), jnp.float32),
        pltpu.VMEM((BPW * NSAMP + 8, ), jnp.float32),
        pltpu.SemaphoreType.DMA,
    ],
    compiler_params=pltpu.CompilerParams(needs_layout_passes=False,
                                         use_tc_tiling_on_sc=False),
)
def _sc_kernel(ctx_f, samp_f, docrows_hbm, w_hbm, ot_hbm, out_hbm, *scratch):
    _sc_body(ctx_f, samp_f, docrows_hbm, w_hbm, ot_hbm, out_hbm, *scratch)


def kernel(doc_ids, context_ids, sample_ids, paragraph_matrix, word_matrix,
           outputs):
    ot = outputs.T
    doc_f = doc_ids.astype(jnp.int32).reshape(B)
    ctx_f = context_ids.astype(jnp.int32).reshape(B * CTX)
    samp_f = sample_ids.astype(jnp.int32).reshape(B * NSAMP)
    pt = paragraph_matrix.T
    docrows = _doc_kernel(pt, doc_f)
    res = _sc_kernel(ctx_f, samp_f, docrows, word_matrix, ot)
    return res.reshape(B, NSAMP)


# speedup vs baseline: 25.9466x; 1.0844x over previous
import functools

import jax
import jax.numpy as jnp
from jax import lax
from jax.experimental import pallas as pl
from jax.experimental.pallas import tpu as pltpu
from jax.experimental.pallas import tpu_sc as plsc

VEC = 64
B = 4096
CTX = 20
NSAMP = 10
NC, NS = 2, 16
NW = NC * NS
BPW = B // NW
LANES = 16
NV = VEC // LANES


_DW = 4


def _doc_body(pt_hbm, doc_f, out_hbm, doc_idx, slab0, slab1, docbuf, sem):
    wid = lax.axis_index("s") * NC + lax.axis_index("c")
    pltpu.sync_copy(doc_f.at[pl.ds(wid * BPW, BPW)], doc_idx)

    lane = lax.iota(jnp.int32, LANES)
    nwaves = BPW // _DW
    slabs = [slab0, slab1]
    pend = [None, None]

    def fire(w):
        jv = doc_idx[pl.ds((w * _DW // LANES) * LANES, LANES)]
        buf = slabs[w % 2]
        cps = []
        for t in range(_DW):
            off = pl.multiple_of(
                (jv[(w * _DW + t) % LANES] >> 7) << 7, 128)
            cps.append(pltpu.async_copy(pt_hbm.at[:, pl.ds(off, 128)],
                                        buf.at[:, pl.ds(t * 128, 128)], sem))
        return cps

    def extract(w):
        jv = doc_idx[pl.ds((w * _DW // LANES) * LANES, LANES)]
        buf = slabs[w % 2]
        for t in range(_DW):
            sid = jv[(w * _DW + t) % LANES]
            col = jnp.full((LANES,), 0, jnp.int32) + (t * 128 + (sid & 127))
            dstrow = jnp.full((LANES,), w * _DW + t, jnp.int32)
            for k in range(NV):
                rows = k * LANES + lane
                v = plsc.load_gather(buf, [rows, col])
                plsc.store_scatter(docbuf, [dstrow, rows], v)

    pend[0] = fire(0)
    for w in range(nwaves):
        if w + 1 < nwaves:
            pend[(w + 1) % 2] = fire(w + 1)
        for cp in pend[w % 2]:
            cp.wait()
        extract(w)

    pltpu.sync_copy(docbuf, out_hbm.at[wid])


@functools.partial(
    pl.kernel,
    out_type=jax.ShapeDtypeStruct((NW, BPW, VEC), jnp.float32),
    mesh=plsc.VectorSubcoreMesh(core_axis_name="c", subcore_axis_name="s"),
    scratch_types=[
        pltpu.VMEM((BPW,), jnp.int32),
        pltpu.VMEM((VEC, _DW * 128), jnp.float32),
        pltpu.VMEM((VEC, _DW * 128), jnp.float32),
        pltpu.VMEM((BPW, VEC), jnp.float32),
        pltpu.SemaphoreType.DMA,
    ],
    compiler_params=pltpu.CompilerParams(needs_layout_passes=False,
                                         use_tc_tiling_on_sc=True),
)
def _doc_kernel(pt_hbm, doc_f, out_hbm, *scratch):
    _doc_body(pt_hbm, doc_f, out_hbm, *scratch)


_CTX_CHUNK = 64
_N_CTX_CHUNKS = BPW // _CTX_CHUNK
_CTX_ROWS = _CTX_CHUNK * CTX


def _sc_body(ctx_w, samp_f, docrows_hbm, w_hbm, ot_hbm, out_hbm,
             ctx_idx, samp_idx, big, inp, res, sem):
    wid = lax.axis_index("s") * NC + lax.axis_index("c")

    pltpu.sync_copy(ctx_w.at[pl.ds(wid * (CTX * BPW), CTX * BPW)],
                    ctx_idx)
    pltpu.sync_copy(samp_f.at[pl.ds(wid * (NSAMP * BPW), NSAMP * BPW)],
                    samp_idx)

    pltpu.sync_copy(docrows_hbm.at[wid], inp)
    cps = [
        pltpu.async_copy(w_hbm.at[ctx_idx.at[pl.ds(c * BPW, BPW)]],
                         inp, sem, add=True)
        for c in range(CTX)
    ]
    cps += [
        pltpu.async_copy(ot_hbm.at[samp_idx.at[pl.ds(j * 128, 128)]],
                         big.at[pl.ds(j * 128, 128)], sem)
        for j in range(10)
    ]
    for cp in cps:
        cp.wait()

    def score_body(i, _):
        lane = lax.iota(jnp.int32, LANES)
        smask = lane < NSAMP
        rows = i * NSAMP + lane
        vin = [inp[i, pl.ds(k * LANES, LANES)] for k in range(NV)]
        acc = jnp.zeros((LANES,), jnp.float32)
        for d in range(VEC):
            col = jnp.full((LANES,), d, jnp.int32)
            g = plsc.load_gather(big, [rows, col], mask=smask)
            acc = acc + vin[d // LANES][d % LANES] * g
        plsc.store_compressed(res.at[pl.ds(i * NSAMP, LANES)], acc, mask=smask)
        return 0

    lax.fori_loop(0, BPW, score_body, 0)

    pltpu.sync_copy(res.at[pl.ds(0, BPW * NSAMP)],
                    out_hbm.at[pl.ds(wid * (BPW * NSAMP), BPW * NSAMP)])


@functools.partial(
    pl.kernel,
    out_type=jax.ShapeDtypeStruct((B * NSAMP,), jnp.float32),
    mesh=plsc.VectorSubcoreMesh(core_axis_name="c", subcore_axis_name="s"),
    scratch_types=[
        pltpu.VMEM((CTX * BPW,), jnp.int32),
        pltpu.VMEM((NSAMP * BPW,), jnp.int32),
        pltpu.VMEM((NSAMP * BPW + 8, VEC), jnp.float32),
        pltpu.VMEM((BPW, VEC), jnp.float32),
        pltpu.VMEM((BPW * NSAMP + 8, ), jnp.float32),
        pltpu.SemaphoreType.DMA,
    ],
    compiler_params=pltpu.CompilerParams(needs_layout_passes=False,
                                         use_tc_tiling_on_sc=False),
)
def _sc_kernel(ctx_w, samp_f, docrows_hbm, w_hbm, ot_hbm, out_hbm, *scratch):
    _sc_body(ctx_w, samp_f, docrows_hbm, w_hbm, ot_hbm, out_hbm, *scratch)


def kernel(doc_ids, context_ids, sample_ids, paragraph_matrix, word_matrix,
           outputs):
    ot = outputs.T
    doc_f = doc_ids.astype(jnp.int32).reshape(B)
    ctx_w = (context_ids.astype(jnp.int32)
             .reshape(NW, BPW, CTX).transpose(0, 2, 1).reshape(B * CTX))
    samp_f = sample_ids.astype(jnp.int32).reshape(B * NSAMP)
    pt = paragraph_matrix.T
    docrows = _doc_kernel(pt, doc_f)
    res = _sc_kernel(ctx_w, samp_f, docrows, word_matrix, ot)
    return res.reshape(B, NSAMP)
